# baseline (speedup 1.0000x reference)
import jax
import jax.numpy as jnp
from jax.experimental import pallas as pl
from jax.experimental.pallas import tpu as pltpu

N = 2048
HD = 64
NREL = 257
BQ = 8
NSTEPS = N // BQ
K = 4
P2LEN = 2184


def _body(delta_ref, edges_ref, tpair0_ref, tpair1_ref, out_ref,
          p2_ref, ring_ref, sems):
    i = pl.program_id(0)

    @pl.when(i == 0)
    def _init():
        for ph in range(2):
            p2_ref[ph, 0:1024, :] = jnp.broadcast_to(edges_ref[0:1, :],
                                                     (1024, 128))
            p2_ref[ph, 1152:P2LEN, :] = jnp.broadcast_to(
                edges_ref[1:2, :], (P2LEN - 1152, 128))
        p2_ref[0, 1024:1152, :] = tpair0_ref[:, :]
        p2_ref[1, 1024:1152, :] = tpair1_ref[:, :]

    slot = jax.lax.rem(i, K)

    @pl.when(i >= K)
    def _reclaim():
        pltpu.make_async_copy(
            ring_ref.at[slot],
            out_ref.at[pl.ds((i - K) * BQ, BQ)],
            sems.at[slot]).wait()

    d = delta_ref[0]
    base = i * BQ
    for r in range(BQ):
        s = 2176 + d - (base + r)
        ph = jax.lax.rem(s, 2)
        r0 = jax.lax.div(s - ph, 2)
        ring_ref[slot, r] = p2_ref[ph, pl.ds(r0, 1024), :]

    pltpu.make_async_copy(
        ring_ref.at[slot],
        out_ref.at[pl.ds(base, BQ)],
        sems.at[slot]).start()

    @pl.when(i == NSTEPS - 1)
    def _drain():
        for k in range(K):
            pltpu.make_async_copy(
                ring_ref.at[k],
                out_ref.at[pl.ds((i - (K - 1) + k) * BQ, BQ)],
                sems.at[k]).wait()


def kernel(len_q, len_k, embedding_table):
    delta = (jnp.asarray(len_k, jnp.int32)
             - jnp.asarray(len_q, jnp.int32)).reshape(1)
    t = embedding_table
    edges = jnp.stack([jnp.concatenate([t[0], t[0]]),
                       jnp.concatenate([t[NREL - 1], t[NREL - 1]])])
    tpair0 = t[0:256].reshape(128, 128)
    tpair1 = t[1:257].reshape(128, 128)
    out = pl.pallas_call(
        _body,
        grid=(NSTEPS,),
        in_specs=[
            pl.BlockSpec(memory_space=pltpu.SMEM),
            pl.BlockSpec((2, 128), lambda i: (0, 0)),
            pl.BlockSpec((128, 128), lambda i: (0, 0)),
            pl.BlockSpec((128, 128), lambda i: (0, 0)),
        ],
        out_specs=pl.BlockSpec(memory_space=pl.ANY),
        out_shape=jax.ShapeDtypeStruct((N, 1024, 128), jnp.float32),
        scratch_shapes=[
            pltpu.VMEM((2, P2LEN, 128), jnp.float32),
            pltpu.VMEM((K, BQ, 1024, 128), jnp.float32),
            pltpu.SemaphoreType.DMA((K,)),
        ],
    )(delta, edges, tpair0, tpair1)
    return out.reshape(N, N, HD)


# speedup vs baseline: 1.0020x; 1.0020x over previous
import jax
import jax.numpy as jnp
from jax.experimental import pallas as pl
from jax.experimental.pallas import tpu as pltpu

N = 2048
HD = 64
NREL = 257
BQ = 16
P2LEN = 2184


def _body(delta_ref, edges_ref, tpair0_ref, tpair1_ref, out_ref, p2_ref):
    @pl.when(pl.program_id(0) == 0)
    def _init():
        for ph in range(2):
            p2_ref[ph, 0:1024, :] = jnp.broadcast_to(edges_ref[0:1, :],
                                                     (1024, 128))
            p2_ref[ph, 1152:P2LEN, :] = jnp.broadcast_to(
                edges_ref[1:2, :], (P2LEN - 1152, 128))
        p2_ref[0, 1024:1152, :] = tpair0_ref[:, :]
        p2_ref[1, 1024:1152, :] = tpair1_ref[:, :]

    d = delta_ref[0]
    base = pl.program_id(0) * BQ
    for r in range(BQ):
        s = 2176 + d - (base + r)
        ph = jax.lax.rem(s, 2)
        r0 = jax.lax.div(s - ph, 2)
        out_ref[r, :, :] = p2_ref[ph, pl.ds(r0, 1024), :]


def kernel(len_q, len_k, embedding_table):
    delta = (jnp.asarray(len_k, jnp.int32)
             - jnp.asarray(len_q, jnp.int32)).reshape(1)
    t = embedding_table
    edges = jnp.stack([jnp.concatenate([t[0], t[0]]),
                       jnp.concatenate([t[NREL - 1], t[NREL - 1]])])
    tpair0 = t[0:256].reshape(128, 128)
    tpair1 = t[1:257].reshape(128, 128)
    out = pl.pallas_call(
        _body,
        grid=(N // BQ,),
        in_specs=[
            pl.BlockSpec(memory_space=pltpu.SMEM),
            pl.BlockSpec((2, 128), lambda i: (0, 0)),
            pl.BlockSpec((128, 128), lambda i: (0, 0)),
            pl.BlockSpec((128, 128), lambda i: (0, 0)),
        ],
        out_specs=pl.BlockSpec((BQ, 1024, 128), lambda i: (i, 0, 0)),
        out_shape=jax.ShapeDtypeStruct((N, 1024, 128), jnp.float32),
        scratch_shapes=[pltpu.VMEM((2, P2LEN, 128), jnp.float32)],
    )(delta, edges, tpair0, tpair1)
    return out.reshape(N, N, HD)
